# baseline (speedup 1.0000x reference)
import functools

import jax
import jax.numpy as jnp
from jax import lax
from jax.experimental import pallas as pl
from jax.experimental.pallas import tpu as pltpu
from jax.experimental.pallas import tpu_sc as plsc

N_OBS = 3276800
TABLE = 1000000
NUM_CORES = 2
NUM_SUBCORES = 16
NUM_WORKERS = NUM_CORES * NUM_SUBCORES
PER_WORKER = N_OBS // NUM_WORKERS
CHUNK = 5120
NUM_CHUNKS = PER_WORKER // CHUNK
LANES = 16
NBUF_IN = 3
NBUF = 2
NPIECES = -(-TABLE // (NUM_SUBCORES * CHUNK))


def _sc_logistic(x, word_ids, a_b, random_x0):
    mesh = plsc.VectorSubcoreMesh(core_axis_name="c", subcore_axis_name="s")

    scratch = [
        pltpu.VMEM_SHARED((TABLE,), jnp.float32),
        pltpu.SemaphoreType.DMA,
        pltpu.VMEM((3 * LANES,), jnp.float32),
        pltpu.SemaphoreType.DMA,
    ]
    for _ in range(NBUF_IN):
        scratch += [
            pltpu.VMEM((CHUNK,), jnp.int32),
            pltpu.VMEM((CHUNK,), jnp.float32),
            pltpu.SemaphoreType.DMA,
        ]
    for _ in range(NBUF):
        scratch += [
            pltpu.VMEM((CHUNK,), jnp.float32),
            pltpu.VMEM((CHUNK,), jnp.float32),
            pltpu.SemaphoreType.DMA,
            pltpu.SemaphoreType.DMA,
        ]

    @functools.partial(
        pl.kernel,
        out_type=jax.ShapeDtypeStruct((N_OBS,), jnp.float32),
        mesh=mesh,
        scratch_types=scratch,
    )
    def run(x_hbm, ids_hbm, abl_hbm, table_hbm, out_hbm,
            tbl_sp, tsem, abl_v, ssem, *bufs):
        idx_v = [bufs[3 * i + 0] for i in range(NBUF_IN)]
        x_v = [bufs[3 * i + 1] for i in range(NBUF_IN)]
        isem = [bufs[3 * i + 2] for i in range(NBUF_IN)]
        gb = bufs[3 * NBUF_IN:]
        val_v = [gb[4 * i + 0] for i in range(NBUF)]
        o_v = [gb[4 * i + 1] for i in range(NBUF)]
        gsem = [gb[4 * i + 2] for i in range(NBUF)]
        wsem = [gb[4 * i + 3] for i in range(NBUF)]

        sid = lax.axis_index("s")
        wid = sid * NUM_CORES + lax.axis_index("c")
        base = wid * PER_WORKER

        def tload(p):
            poff = jnp.minimum((p * NUM_SUBCORES + sid) * CHUNK,
                               TABLE - CHUNK)
            return poff, pltpu.async_copy(
                table_hbm.at[pl.ds(poff, CHUNK)], val_v[p % NBUF],
                gsem[p % NBUF])

        tloads = {0: tload(0)}
        scopy = pltpu.async_copy(abl_hbm, abl_v, ssem)

        def stage_in(c):
            bi = c % NBUF_IN
            off = base + c * CHUNK
            h1 = pltpu.async_copy(
                ids_hbm.at[pl.ds(off, CHUNK)], idx_v[bi], isem[bi])
            h2 = pltpu.async_copy(
                x_hbm.at[pl.ds(off, CHUNK)], x_v[bi], isem[bi])
            return (h1, h2)

        ins = {}
        for c in range(min(NBUF_IN, NUM_CHUNKS)):
            ins[c] = stage_in(c)

        tstores = {}
        for p in range(NPIECES):
            poff, h = tloads[p]
            h.wait()
            tstores[p] = pltpu.async_copy(
                val_v[p % NBUF], tbl_sp.at[pl.ds(poff, CHUNK)], tsem)
            if p + 1 < NPIECES:
                if p - 1 >= 0:
                    tstores[p - 1].wait()
                tloads[p + 1] = tload(p + 1)
        tstores[NPIECES - 1].wait()
        if NPIECES >= 2:
            tstores[NPIECES - 2].wait()
        plsc.subcore_barrier()

        scopy.wait()
        av = abl_v[pl.ds(0, LANES)]
        bv = abl_v[pl.ds(LANES, LANES)]
        lv = abl_v[pl.ds(2 * LANES, LANES)]

        def gather(c):
            bi = c % NBUF_IN
            bg = c % NBUF
            return pltpu.async_copy(
                tbl_sp.at[idx_v[bi]], val_v[bg], gsem[bg])

        gathers = {}
        writes = {}
        ins[0][0].wait()
        ins[0][1].wait()
        gathers[0] = gather(0)

        for c in range(NUM_CHUNKS):
            bb = c % NBUF
            if c + 1 < NUM_CHUNKS:
                ins[c + 1][0].wait()
                ins[c + 1][1].wait()
                gathers[c + 1] = gather(c + 1)
            gathers[c].wait()
            if c - NBUF >= 0:
                writes[c - NBUF].wait()

            bi = c % NBUF_IN

            @plsc.parallel_loop(0, CHUNK, LANES)
            def _vec(i):
                s = pl.ds(i, LANES)
                arg = av * (x_v[bi][s] - val_v[bb][s]) + bv
                o_v[bb][s] = lv / (1.0 + jnp.exp(arg))

            off = base + c * CHUNK
            writes[c] = pltpu.async_copy(
                o_v[bb], out_hbm.at[pl.ds(off, CHUNK)], wsem[bb])
            if c + NBUF_IN < NUM_CHUNKS:
                ins[c + NBUF_IN] = stage_in(c + NBUF_IN)

        for c in range(max(0, NUM_CHUNKS - NBUF), NUM_CHUNKS):
            writes[c].wait()

    return run(x, word_ids, a_b, random_x0)


def kernel(x, word_ids, fixed_L, fixed_x0, fixed_k, random_x0):
    ids = word_ids.astype(jnp.int32)
    k = jnp.asarray(fixed_k, jnp.float32)
    x0 = jnp.asarray(fixed_x0, jnp.float32)
    abl = jnp.concatenate([
        jnp.broadcast_to(-k, (LANES,)),
        jnp.broadcast_to(k * x0, (LANES,)),
        jnp.broadcast_to(jnp.asarray(fixed_L, jnp.float32), (LANES,)),
    ])
    return _sc_logistic(x, ids, abl, random_x0)


# speedup vs baseline: 1.1045x; 1.1045x over previous
import functools

import jax
import jax.numpy as jnp
from jax import lax
from jax.experimental import pallas as pl
from jax.experimental.pallas import tpu as pltpu
from jax.experimental.pallas import tpu_sc as plsc

N_OBS = 3276800
TABLE = 1000000
NUM_CORES = 2
NUM_SUBCORES = 16
NUM_WORKERS = NUM_CORES * NUM_SUBCORES
PER_WORKER = N_OBS // NUM_WORKERS
CHUNK = 6400
NUM_CHUNKS = PER_WORKER // CHUNK
LANES = 16
NBUF_IN = 3
NBUF = 2
NPIECES = -(-TABLE // (NUM_SUBCORES * CHUNK))


def _sc_logistic(x, word_ids, a_b, random_x0):
    mesh = plsc.VectorSubcoreMesh(core_axis_name="c", subcore_axis_name="s")

    scratch = [
        pltpu.VMEM_SHARED((TABLE,), jnp.float32),
        pltpu.SemaphoreType.DMA,
        pltpu.VMEM((3 * LANES,), jnp.float32),
        pltpu.SemaphoreType.DMA,
    ]
    for _ in range(NBUF_IN):
        scratch += [
            pltpu.VMEM((CHUNK,), jnp.int32),
            pltpu.VMEM((CHUNK,), jnp.float32),
            pltpu.SemaphoreType.DMA,
        ]
    for _ in range(NBUF):
        scratch += [
            pltpu.VMEM((CHUNK,), jnp.float32),
            pltpu.VMEM((CHUNK,), jnp.float32),
            pltpu.SemaphoreType.DMA,
            pltpu.SemaphoreType.DMA,
        ]

    @functools.partial(
        pl.kernel,
        out_type=jax.ShapeDtypeStruct((N_OBS,), jnp.float32),
        mesh=mesh,
        scratch_types=scratch,
    )
    def run(x_hbm, ids_hbm, abl_hbm, table_hbm, out_hbm,
            tbl_sp, tsem, abl_v, ssem, *bufs):
        idx_v = [bufs[3 * i + 0] for i in range(NBUF_IN)]
        x_v = [bufs[3 * i + 1] for i in range(NBUF_IN)]
        isem = [bufs[3 * i + 2] for i in range(NBUF_IN)]
        gb = bufs[3 * NBUF_IN:]
        val_v = [gb[4 * i + 0] for i in range(NBUF)]
        o_v = [gb[4 * i + 1] for i in range(NBUF)]
        gsem = [gb[4 * i + 2] for i in range(NBUF)]
        wsem = [gb[4 * i + 3] for i in range(NBUF)]

        sid = lax.axis_index("s")
        wid = sid * NUM_CORES + lax.axis_index("c")
        base = wid * PER_WORKER

        ring = [(val_v[0], gsem[0]), (val_v[1], gsem[1]),
                (o_v[0], wsem[0]), (o_v[1], wsem[1])]
        RD = len(ring)

        def tload(p):
            poff = jnp.minimum((p * NUM_SUBCORES + sid) * CHUNK,
                               TABLE - CHUNK)
            buf, sem = ring[p % RD]
            return poff, buf, pltpu.async_copy(
                table_hbm.at[pl.ds(poff, CHUNK)], buf, sem)

        tloads = {}
        for p in range(min(RD - 1, NPIECES)):
            tloads[p] = tload(p)
        scopy = pltpu.async_copy(abl_hbm, abl_v, ssem)

        def stage_in(c):
            bi = c % NBUF_IN
            off = base + c * CHUNK
            h1 = pltpu.async_copy(
                ids_hbm.at[pl.ds(off, CHUNK)], idx_v[bi], isem[bi])
            h2 = pltpu.async_copy(
                x_hbm.at[pl.ds(off, CHUNK)], x_v[bi], isem[bi])
            return (h1, h2)

        ins = {}
        for c in range(min(NBUF_IN, NUM_CHUNKS)):
            ins[c] = stage_in(c)

        tstores = {}
        for p in range(NPIECES):
            poff, buf, h = tloads[p]
            h.wait()
            tstores[p] = pltpu.async_copy(
                buf, tbl_sp.at[pl.ds(poff, CHUNK)], tsem)
            q = p + RD - 1
            if q < NPIECES:
                if p >= 1:
                    tstores[p - 1].wait()
                tloads[q] = tload(q)
        for p in range(max(0, NPIECES - RD), NPIECES):
            tstores[p].wait()
        plsc.subcore_barrier()

        scopy.wait()
        av = abl_v[pl.ds(0, LANES)]
        bv = abl_v[pl.ds(LANES, LANES)]
        lv = abl_v[pl.ds(2 * LANES, LANES)]

        def gather(c):
            bi = c % NBUF_IN
            bg = c % NBUF
            return pltpu.async_copy(
                tbl_sp.at[idx_v[bi]], val_v[bg], gsem[bg])

        gathers = {}
        writes = {}
        ins[0][0].wait()
        ins[0][1].wait()
        gathers[0] = gather(0)

        for c in range(NUM_CHUNKS):
            bb = c % NBUF
            if c + 1 < NUM_CHUNKS:
                ins[c + 1][0].wait()
                ins[c + 1][1].wait()
                gathers[c + 1] = gather(c + 1)
            gathers[c].wait()
            if c - NBUF >= 0:
                writes[c - NBUF].wait()

            bi = c % NBUF_IN

            @plsc.parallel_loop(0, CHUNK, LANES)
            def _vec(i):
                s = pl.ds(i, LANES)
                arg = av * (x_v[bi][s] - val_v[bb][s]) + bv
                o_v[bb][s] = lv / (1.0 + jnp.exp(arg))

            off = base + c * CHUNK
            writes[c] = pltpu.async_copy(
                o_v[bb], out_hbm.at[pl.ds(off, CHUNK)], wsem[bb])
            if c + NBUF_IN < NUM_CHUNKS:
                ins[c + NBUF_IN] = stage_in(c + NBUF_IN)

        for c in range(max(0, NUM_CHUNKS - NBUF), NUM_CHUNKS):
            writes[c].wait()

    return run(x, word_ids, a_b, random_x0)


def kernel(x, word_ids, fixed_L, fixed_x0, fixed_k, random_x0):
    ids = word_ids.astype(jnp.int32)
    k = jnp.asarray(fixed_k, jnp.float32)
    x0 = jnp.asarray(fixed_x0, jnp.float32)
    abl = jnp.concatenate([
        jnp.broadcast_to(-k, (LANES,)),
        jnp.broadcast_to(k * x0, (LANES,)),
        jnp.broadcast_to(jnp.asarray(fixed_L, jnp.float32), (LANES,)),
    ])
    return _sc_logistic(x, ids, abl, random_x0)


# speedup vs baseline: 1.1085x; 1.0036x over previous
import functools

import jax
import jax.numpy as jnp
from jax import lax
from jax.experimental import pallas as pl
from jax.experimental.pallas import tpu as pltpu
from jax.experimental.pallas import tpu_sc as plsc

N_OBS = 3276800
TABLE = 1000000
NUM_CORES = 2
NUM_SUBCORES = 16
NUM_WORKERS = NUM_CORES * NUM_SUBCORES
PER_WORKER = N_OBS // NUM_WORKERS
CHUNK = 6400
NUM_CHUNKS = PER_WORKER // CHUNK
LANES = 16
NBUF_IN = 3
NBUF = 2
NPIECES = -(-TABLE // (NUM_SUBCORES * CHUNK))


def _sc_logistic(x, word_ids, a_b, random_x0):
    mesh = plsc.VectorSubcoreMesh(core_axis_name="c", subcore_axis_name="s")

    scratch = [
        pltpu.VMEM_SHARED((TABLE,), jnp.float32),
        pltpu.SemaphoreType.DMA,
        pltpu.VMEM((3 * LANES,), jnp.float32),
        pltpu.SemaphoreType.DMA,
    ]
    for _ in range(NBUF_IN):
        scratch += [
            pltpu.VMEM((CHUNK,), jnp.int32),
            pltpu.VMEM((CHUNK,), jnp.float32),
            pltpu.SemaphoreType.DMA,
        ]
    for _ in range(NBUF):
        scratch += [
            pltpu.VMEM((CHUNK,), jnp.float32),
            pltpu.VMEM((CHUNK,), jnp.float32),
            pltpu.SemaphoreType.DMA,
            pltpu.SemaphoreType.DMA,
        ]

    @functools.partial(
        pl.kernel,
        out_type=jax.ShapeDtypeStruct((N_OBS,), jnp.float32),
        mesh=mesh,
        scratch_types=scratch,
    )
    def run(x_hbm, ids_hbm, abl_hbm, table_hbm, out_hbm,
            tbl_sp, tsem, abl_v, ssem, *bufs):
        idx_v = [bufs[3 * i + 0] for i in range(NBUF_IN)]
        x_v = [bufs[3 * i + 1] for i in range(NBUF_IN)]
        isem = [bufs[3 * i + 2] for i in range(NBUF_IN)]
        gb = bufs[3 * NBUF_IN:]
        val_v = [gb[4 * i + 0] for i in range(NBUF)]
        o_v = [gb[4 * i + 1] for i in range(NBUF)]
        gsem = [gb[4 * i + 2] for i in range(NBUF)]
        wsem = [gb[4 * i + 3] for i in range(NBUF)]

        sid = lax.axis_index("s")
        wid = sid * NUM_CORES + lax.axis_index("c")
        base = wid * PER_WORKER

        ring = [(val_v[0], gsem[0]), (val_v[1], gsem[1]),
                (o_v[0], wsem[0]), (o_v[1], wsem[1])]
        RD = len(ring)

        def tload(p):
            poff = jnp.minimum((p * NUM_SUBCORES + sid) * CHUNK,
                               TABLE - CHUNK)
            buf, sem = ring[p % RD]
            return poff, buf, pltpu.async_copy(
                table_hbm.at[pl.ds(poff, CHUNK)], buf, sem)

        tloads = {}
        for p in range(min(RD - 1, NPIECES)):
            tloads[p] = tload(p)
        scopy = pltpu.async_copy(abl_hbm, abl_v, ssem)

        def stage_in(c):
            bi = c % NBUF_IN
            off = base + c * CHUNK
            h1 = pltpu.async_copy(
                ids_hbm.at[pl.ds(off, CHUNK)], idx_v[bi], isem[bi])
            h2 = pltpu.async_copy(
                x_hbm.at[pl.ds(off, CHUNK)], x_v[bi], isem[bi])
            return (h1, h2)

        ins = {}
        for c in range(min(NBUF_IN, NUM_CHUNKS)):
            ins[c] = stage_in(c)

        tstores = {}
        for p in range(NPIECES):
            poff, buf, h = tloads[p]
            h.wait()
            tstores[p] = pltpu.async_copy(
                buf, tbl_sp.at[pl.ds(poff, CHUNK)], tsem)
            q = p + RD - 1
            if q < NPIECES:
                if p >= 1:
                    tstores[p - 1].wait()
                tloads[q] = tload(q)
        for p in range(max(0, NPIECES - RD), NPIECES):
            tstores[p].wait()
        plsc.subcore_barrier()

        scopy.wait()
        av = abl_v[pl.ds(0, LANES)]
        bv = abl_v[pl.ds(LANES, LANES)]
        lv = abl_v[pl.ds(2 * LANES, LANES)]

        def gather(c):
            bi = c % NBUF_IN
            bg = c % NBUF
            return pltpu.async_copy(
                tbl_sp.at[idx_v[bi]], val_v[bg], gsem[bg])

        gathers = {}
        writes = {}
        ins[0][0].wait()
        ins[0][1].wait()
        gathers[0] = gather(0)

        for c in range(NUM_CHUNKS):
            bb = c % NBUF
            if c + 1 < NUM_CHUNKS:
                ins[c + 1][0].wait()
                ins[c + 1][1].wait()
                gathers[c + 1] = gather(c + 1)
            gathers[c].wait()
            if c - NBUF >= 0:
                writes[c - NBUF].wait()

            bi = c % NBUF_IN

            @plsc.parallel_loop(0, CHUNK, LANES, unroll=4)
            def _vec(i):
                s = pl.ds(i, LANES)
                arg = av * (x_v[bi][s] - val_v[bb][s]) + bv
                o_v[bb][s] = lv / (1.0 + jnp.exp(arg))

            off = base + c * CHUNK
            writes[c] = pltpu.async_copy(
                o_v[bb], out_hbm.at[pl.ds(off, CHUNK)], wsem[bb])
            if c + NBUF_IN < NUM_CHUNKS:
                ins[c + NBUF_IN] = stage_in(c + NBUF_IN)

        for c in range(max(0, NUM_CHUNKS - NBUF), NUM_CHUNKS):
            writes[c].wait()

    return run(x, word_ids, a_b, random_x0)


def kernel(x, word_ids, fixed_L, fixed_x0, fixed_k, random_x0):
    ids = word_ids.astype(jnp.int32)
    k = jnp.asarray(fixed_k, jnp.float32)
    x0 = jnp.asarray(fixed_x0, jnp.float32)
    abl = jnp.concatenate([
        jnp.broadcast_to(-k, (LANES,)),
        jnp.broadcast_to(k * x0, (LANES,)),
        jnp.broadcast_to(jnp.asarray(fixed_L, jnp.float32), (LANES,)),
    ])
    return _sc_logistic(x, ids, abl, random_x0)


# speedup vs baseline: 1.1228x; 1.0129x over previous
import functools

import jax
import jax.numpy as jnp
from jax import lax
from jax.experimental import pallas as pl
from jax.experimental.pallas import tpu as pltpu
from jax.experimental.pallas import tpu_sc as plsc

N_OBS = 3276800
TABLE = 1000000
NUM_CORES = 2
NUM_SUBCORES = 16
NUM_WORKERS = NUM_CORES * NUM_SUBCORES
PER_WORKER = N_OBS // NUM_WORKERS
CHUNK = 6400
NUM_CHUNKS = PER_WORKER // CHUNK
LANES = 16
NBUF_IN = 3
NBUF = 2
NPIECES = -(-TABLE // (NUM_SUBCORES * CHUNK))


def _sc_logistic(x, word_ids, a_b, random_x0):
    mesh = plsc.VectorSubcoreMesh(core_axis_name="c", subcore_axis_name="s")

    scratch = [
        pltpu.VMEM_SHARED((TABLE,), jnp.float32),
        pltpu.SemaphoreType.DMA,
        pltpu.VMEM((3 * LANES,), jnp.float32),
        pltpu.SemaphoreType.DMA,
    ]
    for _ in range(NBUF_IN):
        scratch += [
            pltpu.VMEM((CHUNK,), jnp.int32),
            pltpu.VMEM((CHUNK,), jnp.float32),
            pltpu.SemaphoreType.DMA,
        ]
    for _ in range(NBUF):
        scratch += [
            pltpu.VMEM((CHUNK,), jnp.float32),
            pltpu.VMEM((CHUNK,), jnp.float32),
            pltpu.SemaphoreType.DMA,
            pltpu.SemaphoreType.DMA,
            pltpu.SemaphoreType.DMA,
        ]

    @functools.partial(
        pl.kernel,
        out_type=jax.ShapeDtypeStruct((N_OBS,), jnp.float32),
        mesh=mesh,
        scratch_types=scratch,
    )
    def run(x_hbm, ids_hbm, abl_hbm, table_hbm, out_hbm,
            tbl_sp, tsem, abl_v, ssem, *bufs):
        idx_v = [bufs[3 * i + 0] for i in range(NBUF_IN)]
        x_v = [bufs[3 * i + 1] for i in range(NBUF_IN)]
        isem = [bufs[3 * i + 2] for i in range(NBUF_IN)]
        gb = bufs[3 * NBUF_IN:]
        val_v = [gb[5 * i + 0] for i in range(NBUF)]
        o_v = [gb[5 * i + 1] for i in range(NBUF)]
        gsem = [gb[5 * i + 2] for i in range(NBUF)]
        g2sem = [gb[5 * i + 3] for i in range(NBUF)]
        wsem = [gb[5 * i + 4] for i in range(NBUF)]

        sid = lax.axis_index("s")
        wid = sid * NUM_CORES + lax.axis_index("c")
        base = wid * PER_WORKER

        ring = [(val_v[0], gsem[0]), (val_v[1], gsem[1]),
                (o_v[0], wsem[0]), (o_v[1], wsem[1])]
        RD = len(ring)

        def tload(p):
            poff = jnp.minimum((p * NUM_SUBCORES + sid) * CHUNK,
                               TABLE - CHUNK)
            buf, sem = ring[p % RD]
            return poff, buf, pltpu.async_copy(
                table_hbm.at[pl.ds(poff, CHUNK)], buf, sem)

        tloads = {}
        for p in range(min(RD - 1, NPIECES)):
            tloads[p] = tload(p)
        scopy = pltpu.async_copy(abl_hbm, abl_v, ssem)

        def stage_in(c):
            bi = c % NBUF_IN
            off = base + c * CHUNK
            h1 = pltpu.async_copy(
                ids_hbm.at[pl.ds(off, CHUNK)], idx_v[bi], isem[bi])
            h2 = pltpu.async_copy(
                x_hbm.at[pl.ds(off, CHUNK)], x_v[bi], isem[bi])
            return (h1, h2)

        ins = {}
        for c in range(min(NBUF_IN, NUM_CHUNKS)):
            ins[c] = stage_in(c)

        tstores = {}
        for p in range(NPIECES):
            poff, buf, h = tloads[p]
            h.wait()
            tstores[p] = pltpu.async_copy(
                buf, tbl_sp.at[pl.ds(poff, CHUNK)], tsem)
            q = p + RD - 1
            if q < NPIECES:
                if p >= 1:
                    tstores[p - 1].wait()
                tloads[q] = tload(q)
        for p in range(max(0, NPIECES - RD), NPIECES):
            tstores[p].wait()
        plsc.subcore_barrier()

        scopy.wait()
        av = abl_v[pl.ds(0, LANES)]
        bv = abl_v[pl.ds(LANES, LANES)]
        lv = abl_v[pl.ds(2 * LANES, LANES)]

        H = CHUNK // 2

        def gather(c):
            bi = c % NBUF_IN
            bg = c % NBUF
            h1 = pltpu.async_copy(
                tbl_sp.at[idx_v[bi].at[pl.ds(0, H)]],
                val_v[bg].at[pl.ds(0, H)], gsem[bg])
            h2 = pltpu.async_copy(
                tbl_sp.at[idx_v[bi].at[pl.ds(H, H)]],
                val_v[bg].at[pl.ds(H, H)], g2sem[bg])
            return (h1, h2)

        gathers = {}
        writes = {}
        ins[0][0].wait()
        ins[0][1].wait()
        gathers[0] = gather(0)

        for c in range(NUM_CHUNKS):
            bb = c % NBUF
            if c + 1 < NUM_CHUNKS:
                ins[c + 1][0].wait()
                ins[c + 1][1].wait()
                gathers[c + 1] = gather(c + 1)
            gathers[c][0].wait()
            gathers[c][1].wait()
            if c - NBUF >= 0:
                writes[c - NBUF].wait()

            bi = c % NBUF_IN

            @plsc.parallel_loop(0, CHUNK, LANES, unroll=4)
            def _vec(i):
                s = pl.ds(i, LANES)
                arg = av * (x_v[bi][s] - val_v[bb][s]) + bv
                o_v[bb][s] = lv / (1.0 + jnp.exp(arg))

            off = base + c * CHUNK
            writes[c] = pltpu.async_copy(
                o_v[bb], out_hbm.at[pl.ds(off, CHUNK)], wsem[bb])
            if c + NBUF_IN < NUM_CHUNKS:
                ins[c + NBUF_IN] = stage_in(c + NBUF_IN)

        for c in range(max(0, NUM_CHUNKS - NBUF), NUM_CHUNKS):
            writes[c].wait()

    return run(x, word_ids, a_b, random_x0)


def kernel(x, word_ids, fixed_L, fixed_x0, fixed_k, random_x0):
    ids = word_ids.astype(jnp.int32)
    k = jnp.asarray(fixed_k, jnp.float32)
    x0 = jnp.asarray(fixed_x0, jnp.float32)
    abl = jnp.concatenate([
        jnp.broadcast_to(-k, (LANES,)),
        jnp.broadcast_to(k * x0, (LANES,)),
        jnp.broadcast_to(jnp.asarray(fixed_L, jnp.float32), (LANES,)),
    ])
    return _sc_logistic(x, ids, abl, random_x0)
